# Initial kernel scaffold; baseline (speedup 1.0000x reference)
import jax
import jax.numpy as jnp
from jax.experimental import pallas as pl


def kernel(x, W_router, b_router, W1, b1, W2, b2):
    raise NotImplementedError("write your pallas kernel here")


# speedup vs baseline: 1.7346x; 1.7346x over previous
import functools
import math

import jax
import jax.numpy as jnp
from jax.experimental import pallas as pl
from jax.experimental.pallas import tpu as pltpu

HIDDEN = 1024
NUM_EXPERTS = 8
TOP_K = 2
FFN = HIDDEN * 4

BM = 512
BN = 512
NN = FFN // BN


def _gelu_exact(x):
    return 0.5 * x * (1.0 + jax.lax.erf(x / math.sqrt(2.0)))


def _dense_body(x_ref, wr_ref, br_ref, w1_ref, b1_ref, w2_ref, b2_ref,
                out_ref, w8_scr):
    e = pl.program_id(1)
    n = pl.program_id(2)

    @pl.when((e == 0) & (n == 0))
    def _router():
        logits = jnp.dot(x_ref[...], wr_ref[...].T,
                         preferred_element_type=jnp.float32) + br_ref[...]
        mx = jnp.max(logits, axis=1, keepdims=True)
        ex = jnp.exp(logits - mx)
        probs = ex / jnp.sum(ex, axis=1, keepdims=True)
        m1 = jnp.max(probs, axis=1, keepdims=True)
        iota = jax.lax.broadcasted_iota(jnp.int32, probs.shape, 1)
        i1 = jnp.min(jnp.where(probs == m1, iota, NUM_EXPERTS),
                     axis=1, keepdims=True)
        probs2 = jnp.where(iota == i1, -jnp.inf, probs)
        m2 = jnp.max(probs2, axis=1, keepdims=True)
        i2 = jnp.min(jnp.where(probs2 == m2, iota, NUM_EXPERTS),
                     axis=1, keepdims=True)
        s = m1 + m2
        w8 = (jnp.where(iota == i1, m1 / s, 0.0)
              + jnp.where(iota == i2, m2 / s, 0.0))
        w8_scr[...] = w8

    xb = x_ref[...].astype(jnp.bfloat16)
    w1b = w1_ref[0].astype(jnp.bfloat16)
    h = jnp.dot(xb, w1b.T, preferred_element_type=jnp.float32) + b1_ref[0]
    h = _gelu_exact(h)
    w2b = w2_ref[0].astype(jnp.bfloat16)
    part = jnp.dot(h.astype(jnp.bfloat16), w2b.T,
                   preferred_element_type=jnp.float32)

    w8 = w8_scr[...]
    lane = jax.lax.broadcasted_iota(jnp.int32, w8.shape, 1)
    we = jnp.sum(jnp.where(lane == e, w8, 0.0), axis=1, keepdims=True)

    @pl.when(n == 0)
    def _bias2():
        part_b = part + b2_ref[0]

        @pl.when(e == 0)
        def _init():
            out_ref[...] = we * part_b

        @pl.when(e != 0)
        def _acc():
            out_ref[...] += we * part_b

    @pl.when(n != 0)
    def _accn():
        out_ref[...] += we * part


def _dense_moe(x_flat, W_router, b_router, W1, b1, W2, b2, *, interpret=False):
    T = x_flat.shape[0]
    nm = T // BM
    br2 = b_router.reshape(1, NUM_EXPERTS)
    b1r = b1.reshape(NUM_EXPERTS * NN, 1, BN)
    b2r = b2.reshape(NUM_EXPERTS, 1, HIDDEN)
    grid = (nm, NUM_EXPERTS, NN)
    out = pl.pallas_call(
        _dense_body,
        grid=grid,
        in_specs=[
            pl.BlockSpec((BM, HIDDEN), lambda m, e, n: (m, 0)),
            pl.BlockSpec((NUM_EXPERTS, HIDDEN), lambda m, e, n: (0, 0)),
            pl.BlockSpec((1, NUM_EXPERTS), lambda m, e, n: (0, 0)),
            pl.BlockSpec((1, BN, HIDDEN), lambda m, e, n: (e, n, 0)),
            pl.BlockSpec((1, 1, BN), lambda m, e, n: (e * NN + n, 0, 0)),
            pl.BlockSpec((1, HIDDEN, BN), lambda m, e, n: (e, 0, n)),
            pl.BlockSpec((1, 1, HIDDEN), lambda m, e, n: (e, 0, 0)),
        ],
        out_specs=pl.BlockSpec((BM, HIDDEN), lambda m, e, n: (m, 0)),
        out_shape=jax.ShapeDtypeStruct((T, HIDDEN), jnp.float32),
        scratch_shapes=[pltpu.VMEM((BM, NUM_EXPERTS), jnp.float32)],
        interpret=interpret,
    )(x_flat, W_router, br2, W1, b1r, W2, b2r)
    return out


def kernel(x, W_router, b_router, W1, b1, W2, b2):
    B, S, H = x.shape
    x_flat = x.reshape(-1, H)
    out = _dense_moe(x_flat, W_router, b_router, W1, b1, W2, b2)
    return out.reshape(B, S, H)


# speedup vs baseline: 2.1126x; 1.2179x over previous
import functools
import math

import jax
import jax.numpy as jnp
from jax import lax
from jax.experimental import pallas as pl
from jax.experimental.pallas import tpu as pltpu
from jax.experimental.pallas import tpu_sc as plsc

HIDDEN = 1024
E = 8
TOP_K = 2
FFN = 4096

BM = 512
BN = 512
NN = FFN // BN

NC = 2
NS = 16
NW = NC * NS


def _gelu_exact(x):
    return 0.5 * x * (1.0 + lax.erf(x / math.sqrt(2.0)))


def _router_body(x_ref, wr_ref, br_ref, idx_ref, w_ref):
    logits = jnp.dot(x_ref[...], wr_ref[...].T,
                     preferred_element_type=jnp.float32) + br_ref[...]
    mx = jnp.max(logits, axis=1, keepdims=True)
    ex = jnp.exp(logits - mx)
    probs = ex / jnp.sum(ex, axis=1, keepdims=True)
    iota = lax.broadcasted_iota(jnp.int32, probs.shape, 1)
    m1 = jnp.max(probs, axis=1, keepdims=True)
    i1 = jnp.min(jnp.where(probs == m1, iota, E), axis=1, keepdims=True)
    probs2 = jnp.where(iota == i1, -jnp.inf, probs)
    m2 = jnp.max(probs2, axis=1, keepdims=True)
    i2 = jnp.min(jnp.where(probs2 == m2, iota, E), axis=1, keepdims=True)
    s = m1 + m2
    idx_ref[...] = jnp.concatenate([i1, i2], axis=1)
    w_ref[...] = jnp.concatenate([m1 / s, m2 / s], axis=1)


def _router(x_flat, W_router, b_router):
    t = x_flat.shape[0]
    nm = t // BM
    return pl.pallas_call(
        _router_body,
        grid=(nm,),
        in_specs=[
            pl.BlockSpec((BM, HIDDEN), lambda m: (m, 0)),
            pl.BlockSpec((E, HIDDEN), lambda m: (0, 0)),
            pl.BlockSpec((1, E), lambda m: (0, 0)),
        ],
        out_specs=[
            pl.BlockSpec((BM, TOP_K), lambda m: (m, 0)),
            pl.BlockSpec((BM, TOP_K), lambda m: (m, 0)),
        ],
        out_shape=[
            jax.ShapeDtypeStruct((t, TOP_K), jnp.int32),
            jax.ShapeDtypeStruct((t, TOP_K), jnp.float32),
        ],
    )(x_flat, W_router, b_router.reshape(1, E))


def _make_sc_gather(p_rows):
    rows_per_w = p_rows // NW
    chunk = 96
    nch = rows_per_w // chunk
    assert nch * chunk == rows_per_w

    def body(x_hbm, rt_hbm, xg_hbm, idx_v, rows_v, sem):
        wid = lax.axis_index("s") * NC + lax.axis_index("c")
        base_w = wid * rows_per_w
        for c in range(nch):
            base = base_w + c * chunk
            pltpu.sync_copy(rt_hbm.at[pl.ds(base, chunk)], idx_v)
            pltpu.async_copy(x_hbm.at[idx_v], rows_v, sem).wait()
            pltpu.sync_copy(rows_v, xg_hbm.at[pl.ds(base, chunk)])

    return pl.kernel(
        body,
        out_type=jax.ShapeDtypeStruct((p_rows, HIDDEN), jnp.float32),
        mesh=plsc.VectorSubcoreMesh(core_axis_name="c", subcore_axis_name="s"),
        scratch_types=[
            pltpu.VMEM((chunk,), jnp.int32),
            pltpu.VMEM((chunk, HIDDEN), jnp.float32),
            pltpu.SemaphoreType.DMA,
        ],
    )


def _ffn_body(be_ref, na_ref, xg_ref, w1_ref, b1_ref, w2_ref, b2_ref, ww_ref,
              out_ref):
    b = pl.program_id(0)
    n = pl.program_id(1)

    @pl.when(b < na_ref[0])
    def _():
        xb = xg_ref[...].astype(jnp.bfloat16)
        h = jnp.dot(xb, w1_ref[0].T,
                    preferred_element_type=jnp.float32) + b1_ref[0]
        h = _gelu_exact(h)
        part = jnp.dot(h.astype(jnp.bfloat16), w2_ref[0].T,
                       preferred_element_type=jnp.float32)
        ww = ww_ref[...]

        @pl.when(n == 0)
        def _init():
            out_ref[...] = ww * (part + b2_ref[0])

        @pl.when(n != 0)
        def _acc():
            out_ref[...] += ww * part


def _grouped_ffn(nb, be, na, xg, W1b, b1, W2b, b2, roww):
    p_rows = xg.shape[0]
    b1r = b1.reshape(E * NN, 1, BN)
    b2r = b2.reshape(E, 1, HIDDEN)
    grid_spec = pltpu.PrefetchScalarGridSpec(
        num_scalar_prefetch=2,
        grid=(nb, NN),
        in_specs=[
            pl.BlockSpec((BM, HIDDEN),
                         lambda b, n, be, na: (jnp.minimum(b, na[0] - 1), 0)),
            pl.BlockSpec((1, BN, HIDDEN),
                         lambda b, n, be, na:
                         (be[b], jnp.where(b < na[0], n, NN - 1), 0)),
            pl.BlockSpec((1, 1, BN),
                         lambda b, n, be, na:
                         (be[b] * NN + jnp.where(b < na[0], n, NN - 1), 0, 0)),
            pl.BlockSpec((1, HIDDEN, BN),
                         lambda b, n, be, na:
                         (be[b], 0, jnp.where(b < na[0], n, NN - 1))),
            pl.BlockSpec((1, 1, HIDDEN),
                         lambda b, n, be, na: (be[b], 0, 0)),
            pl.BlockSpec((BM, 1),
                         lambda b, n, be, na: (jnp.minimum(b, na[0] - 1), 0)),
        ],
        out_specs=pl.BlockSpec((BM, HIDDEN), lambda b, n, be, na: (b, 0)),
    )
    return pl.pallas_call(
        _ffn_body,
        grid_spec=grid_spec,
        out_shape=jax.ShapeDtypeStruct((p_rows, HIDDEN), jnp.float32),
    )(be, na, xg, W1b, b1r, W2b, b2r, roww.reshape(p_rows, 1))


def _make_sc_combine(t_tokens):
    toks_per_w = t_tokens // NW
    tch = 32
    nch = toks_per_w // tch
    assert nch * tch == toks_per_w
    nlane = HIDDEN // 16

    def body(yw_hbm, dest_hbm, out_hbm, idx_v, rows_v, out_v, sem):
        wid = lax.axis_index("s") * NC + lax.axis_index("c")
        for c in range(nch):
            tb = wid * toks_per_w + c * tch
            pltpu.sync_copy(dest_hbm.at[pl.ds(2 * tb, 2 * tch)], idx_v)
            pltpu.async_copy(yw_hbm.at[idx_v], rows_v, sem).wait()

            def tok(t, carry):
                for j in range(nlane):
                    sl = pl.ds(j * 16, 16)
                    out_v[t, sl] = rows_v[2 * t, sl] + rows_v[2 * t + 1, sl]
                return carry

            lax.fori_loop(0, tch, tok, 0)
            pltpu.sync_copy(out_v, out_hbm.at[pl.ds(tb, tch)])

    return pl.kernel(
        body,
        out_type=jax.ShapeDtypeStruct((t_tokens, HIDDEN), jnp.float32),
        mesh=plsc.VectorSubcoreMesh(core_axis_name="c", subcore_axis_name="s"),
        scratch_types=[
            pltpu.VMEM((2 * tch,), jnp.int32),
            pltpu.VMEM((2 * tch, HIDDEN), jnp.float32),
            pltpu.VMEM((tch, HIDDEN), jnp.float32),
            pltpu.SemaphoreType.DMA,
        ],
    )


def _dispatch_metadata(top_idx, top_w, nb):
    q = top_idx.size
    e_flat = top_idx.reshape(-1)
    w_flat = top_w.reshape(-1)
    onehot = (e_flat[:, None] == jnp.arange(E, dtype=jnp.int32)[None, :])
    cum = jnp.cumsum(onehot.astype(jnp.int32), axis=0)
    counts = cum[-1]
    rank = jnp.take_along_axis(cum, e_flat[:, None], axis=1)[:, 0] - 1
    aligned = ((counts + BM - 1) // BM) * BM
    offs = jnp.concatenate(
        [jnp.zeros((1,), jnp.int32), jnp.cumsum(aligned)[:-1].astype(jnp.int32)])
    dest = (offs[e_flat] + rank).astype(jnp.int32)
    p_rows = nb * BM
    row_token = jnp.zeros((p_rows,), jnp.int32).at[dest].set(
        jnp.arange(q, dtype=jnp.int32) // TOP_K)
    roww = jnp.zeros((p_rows,), jnp.float32).at[dest].set(w_flat)
    total = offs[-1] + aligned[-1]
    na = (total // BM).astype(jnp.int32)
    starts = jnp.arange(nb, dtype=jnp.int32) * BM
    be = (jnp.searchsorted(offs, starts, side="right") - 1).astype(jnp.int32)
    be_last = jnp.take(be, na - 1)
    be = jnp.where(jnp.arange(nb) < na, be, be_last)
    return dest, row_token, roww, be, na.reshape(1)


def kernel(x, W_router, b_router, W1, b1, W2, b2):
    B, S, H = x.shape
    t_tokens = B * S
    x_flat = x.reshape(t_tokens, H)
    q = t_tokens * TOP_K
    nb = (q + E * (BM - 1) + BM - 1) // BM

    top_idx, top_w = _router(x_flat, W_router, b_router)
    dest, row_token, roww, be, na = _dispatch_metadata(top_idx, top_w, nb)

    xg = _make_sc_gather(nb * BM)(x_flat, row_token)
    W1b = W1.astype(jnp.bfloat16)
    W2b = W2.astype(jnp.bfloat16)
    yw = _grouped_ffn(nb, be, na, xg, W1b, b1, W2b, b2, roww)
    out = _make_sc_combine(t_tokens)(yw, dest)
    return out.reshape(B, S, H)


# speedup vs baseline: 2.5540x; 1.2089x over previous
import functools
import math

import jax
import jax.numpy as jnp
from jax import lax
from jax.experimental import pallas as pl
from jax.experimental.pallas import tpu as pltpu
from jax.experimental.pallas import tpu_sc as plsc

HIDDEN = 1024
E = 8
TOP_K = 2
FFN = 4096

BM = 512
BN = 512
NN = FFN // BN

NC = 2
NS = 16
NW = NC * NS


def _gelu_exact(x):
    return 0.5 * x * (1.0 + lax.erf(x / math.sqrt(2.0)))


def _router_body(x_ref, wr_ref, br_ref, idx_ref, w_ref):
    logits = jnp.dot(x_ref[...], wr_ref[...].T,
                     preferred_element_type=jnp.float32) + br_ref[...]
    mx = jnp.max(logits, axis=1, keepdims=True)
    ex = jnp.exp(logits - mx)
    probs = ex / jnp.sum(ex, axis=1, keepdims=True)
    iota = lax.broadcasted_iota(jnp.int32, probs.shape, 1)
    m1 = jnp.max(probs, axis=1, keepdims=True)
    i1 = jnp.min(jnp.where(probs == m1, iota, E), axis=1, keepdims=True)
    probs2 = jnp.where(iota == i1, -jnp.inf, probs)
    m2 = jnp.max(probs2, axis=1, keepdims=True)
    i2 = jnp.min(jnp.where(probs2 == m2, iota, E), axis=1, keepdims=True)
    s = m1 + m2
    idx_ref[...] = jnp.concatenate([i1, i2], axis=1)
    w_ref[...] = jnp.concatenate([m1 / s, m2 / s], axis=1)


def _router(x_flat, W_router, b_router):
    t = x_flat.shape[0]
    nm = t // BM
    return pl.pallas_call(
        _router_body,
        grid=(nm,),
        in_specs=[
            pl.BlockSpec((BM, HIDDEN), lambda m: (m, 0)),
            pl.BlockSpec((E, HIDDEN), lambda m: (0, 0)),
            pl.BlockSpec((1, E), lambda m: (0, 0)),
        ],
        out_specs=[
            pl.BlockSpec((BM, TOP_K), lambda m: (m, 0)),
            pl.BlockSpec((BM, TOP_K), lambda m: (m, 0)),
        ],
        out_shape=[
            jax.ShapeDtypeStruct((t, TOP_K), jnp.int32),
            jax.ShapeDtypeStruct((t, TOP_K), jnp.float32),
        ],
    )(x_flat, W_router, b_router.reshape(1, E))


def _make_sc_gather(p_rows):
    rows_per_w = p_rows // NW
    chunk = 48
    nch = rows_per_w // chunk
    assert nch * chunk == rows_per_w

    def body(x_hbm, rt_hbm, xg_hbm, idx_v, buf0, buf1,
             gs0, gs1, ss0, ss1):
        wid = lax.axis_index("s") * NC + lax.axis_index("c")
        base_w = wid * rows_per_w
        pltpu.sync_copy(rt_hbm.at[pl.ds(base_w, rows_per_w)], idx_v)
        bufs = (buf0, buf1)
        gsems = (gs0, gs1)
        ssems = (ss0, ss1)

        def start_gather(c, buf, sem):
            return pltpu.async_copy(
                x_hbm.at[idx_v.at[pl.ds(c * chunk, chunk)]], buf, sem)

        g = [None, None]
        s = [None, None]
        g[0] = start_gather(0, bufs[0], gsems[0])
        for c in range(nch):
            cur = c & 1
            g[cur].wait()
            if c + 1 < nch:
                nxt = (c + 1) & 1
                if s[nxt] is not None:
                    s[nxt].wait()
                    s[nxt] = None
                g[nxt] = start_gather(c + 1, bufs[nxt], gsems[nxt])
            s[cur] = pltpu.async_copy(
                bufs[cur], xg_hbm.at[pl.ds(base_w + c * chunk, chunk)],
                ssems[cur])
        for h in s:
            if h is not None:
                h.wait()

    return pl.kernel(
        body,
        out_type=jax.ShapeDtypeStruct((p_rows, HIDDEN), jnp.float32),
        mesh=plsc.VectorSubcoreMesh(core_axis_name="c", subcore_axis_name="s"),
        scratch_types=[
            pltpu.VMEM((rows_per_w,), jnp.int32),
            pltpu.VMEM((chunk, HIDDEN), jnp.float32),
            pltpu.VMEM((chunk, HIDDEN), jnp.float32),
            pltpu.SemaphoreType.DMA,
            pltpu.SemaphoreType.DMA,
            pltpu.SemaphoreType.DMA,
            pltpu.SemaphoreType.DMA,
        ],
    )


def _ffn_body(be_ref, na_ref, xg_ref, w1_ref, b1_ref, w2_ref, b2_ref, ww_ref,
              out_ref):
    b = pl.program_id(0)
    n = pl.program_id(1)

    @pl.when(b < na_ref[0])
    def _():
        xb = xg_ref[...].astype(jnp.bfloat16)
        h = jnp.dot(xb, w1_ref[0].astype(jnp.bfloat16).T,
                    preferred_element_type=jnp.float32) + b1_ref[0]
        h = _gelu_exact(h)
        part = jnp.dot(h.astype(jnp.bfloat16), w2_ref[0].astype(jnp.bfloat16).T,
                       preferred_element_type=jnp.float32)
        ww = ww_ref[...]

        @pl.when(n == 0)
        def _init():
            out_ref[...] = ww * (part + b2_ref[0])

        @pl.when(n != 0)
        def _acc():
            out_ref[...] += ww * part


def _grouped_ffn(nb, be, na, xg, W1b, b1, W2b, b2, roww):
    p_rows = xg.shape[0]
    b1r = b1.reshape(E * NN, 1, BN)
    b2r = b2.reshape(E, 1, HIDDEN)
    grid_spec = pltpu.PrefetchScalarGridSpec(
        num_scalar_prefetch=2,
        grid=(nb, NN),
        in_specs=[
            pl.BlockSpec((BM, HIDDEN),
                         lambda b, n, be, na: (jnp.minimum(b, na[0] - 1), 0)),
            pl.BlockSpec((1, BN, HIDDEN),
                         lambda b, n, be, na:
                         (be[b], jnp.where(b < na[0], n, NN - 1), 0)),
            pl.BlockSpec((1, 1, BN),
                         lambda b, n, be, na:
                         (be[b] * NN + jnp.where(b < na[0], n, NN - 1), 0, 0)),
            pl.BlockSpec((1, HIDDEN, BN),
                         lambda b, n, be, na:
                         (be[b], 0, jnp.where(b < na[0], n, NN - 1))),
            pl.BlockSpec((1, 1, HIDDEN),
                         lambda b, n, be, na: (be[b], 0, 0)),
            pl.BlockSpec((BM, 1),
                         lambda b, n, be, na: (jnp.minimum(b, na[0] - 1), 0)),
        ],
        out_specs=pl.BlockSpec((BM, HIDDEN), lambda b, n, be, na: (b, 0)),
    )
    return pl.pallas_call(
        _ffn_body,
        grid_spec=grid_spec,
        out_shape=jax.ShapeDtypeStruct((p_rows, HIDDEN), jnp.float32),
    )(be, na, xg, W1b, b1r, W2b, b2r, roww.reshape(p_rows, 1))


def _make_sc_combine(t_tokens):
    toks_per_w = t_tokens // NW
    tch = 16
    nch = toks_per_w // tch
    assert nch * tch == toks_per_w
    nlane = HIDDEN // 16

    def body(yw_hbm, dest_hbm, out_hbm, idx_v, r0, r1, o0, o1,
             gs0, gs1, ss0, ss1):
        wid = lax.axis_index("s") * NC + lax.axis_index("c")
        base_t = wid * toks_per_w
        pltpu.sync_copy(dest_hbm.at[pl.ds(2 * base_t, 2 * toks_per_w)], idx_v)
        rbufs = (r0, r1)
        obufs = (o0, o1)
        gsems = (gs0, gs1)
        ssems = (ss0, ss1)

        def start_gather(c, buf, sem):
            return pltpu.async_copy(
                yw_hbm.at[idx_v.at[pl.ds(c * 2 * tch, 2 * tch)]], buf, sem)

        g = [None, None]
        s = [None, None]
        g[0] = start_gather(0, rbufs[0], gsems[0])
        for c in range(nch):
            cur = c & 1
            g[cur].wait()
            if c + 1 < nch:
                nxt = (c + 1) & 1
                g[nxt] = start_gather(c + 1, rbufs[nxt], gsems[nxt])
            if s[cur] is not None:
                s[cur].wait()
            rows_v = rbufs[cur]
            out_v = obufs[cur]

            def tok(t, carry):
                for j in range(nlane):
                    sl = pl.ds(j * 16, 16)
                    out_v[t, sl] = rows_v[2 * t, sl] + rows_v[2 * t + 1, sl]
                return carry

            lax.fori_loop(0, tch, tok, 0)
            s[cur] = pltpu.async_copy(
                out_v, out_hbm.at[pl.ds(base_t + c * tch, tch)], ssems[cur])
        for h in s:
            if h is not None:
                h.wait()

    return pl.kernel(
        body,
        out_type=jax.ShapeDtypeStruct((t_tokens, HIDDEN), jnp.float32),
        mesh=plsc.VectorSubcoreMesh(core_axis_name="c", subcore_axis_name="s"),
        scratch_types=[
            pltpu.VMEM((2 * toks_per_w,), jnp.int32),
            pltpu.VMEM((2 * tch, HIDDEN), jnp.float32),
            pltpu.VMEM((2 * tch, HIDDEN), jnp.float32),
            pltpu.VMEM((tch, HIDDEN), jnp.float32),
            pltpu.VMEM((tch, HIDDEN), jnp.float32),
            pltpu.SemaphoreType.DMA,
            pltpu.SemaphoreType.DMA,
            pltpu.SemaphoreType.DMA,
            pltpu.SemaphoreType.DMA,
        ],
    )


def _dispatch_metadata(top_idx, top_w, nb):
    q = top_idx.size
    e_flat = top_idx.reshape(-1)
    w_flat = top_w.reshape(-1)
    onehot = (e_flat[:, None] == jnp.arange(E, dtype=jnp.int32)[None, :])
    cum = jnp.cumsum(onehot.astype(jnp.int32), axis=0)
    counts = cum[-1]
    rank = jnp.take_along_axis(cum, e_flat[:, None], axis=1)[:, 0] - 1
    aligned = ((counts + BM - 1) // BM) * BM
    offs = jnp.concatenate(
        [jnp.zeros((1,), jnp.int32), jnp.cumsum(aligned)[:-1].astype(jnp.int32)])
    dest = (offs[e_flat] + rank).astype(jnp.int32)
    p_rows = nb * BM
    row_token = jnp.zeros((p_rows,), jnp.int32).at[dest].set(
        jnp.arange(q, dtype=jnp.int32) // TOP_K)
    roww = jnp.zeros((p_rows,), jnp.float32).at[dest].set(w_flat)
    total = offs[-1] + aligned[-1]
    na = (total // BM).astype(jnp.int32)
    starts = jnp.arange(nb, dtype=jnp.int32) * BM
    be = (jnp.searchsorted(offs, starts, side="right") - 1).astype(jnp.int32)
    be_last = jnp.take(be, na - 1)
    be = jnp.where(jnp.arange(nb) < na, be, be_last)
    return dest, row_token, roww, be, na.reshape(1)


def kernel(x, W_router, b_router, W1, b1, W2, b2):
    B, S, H = x.shape
    t_tokens = B * S
    x_flat = x.reshape(t_tokens, H)
    q = t_tokens * TOP_K
    nb = (q + E * (BM - 1) + BM - 1) // BM

    top_idx, top_w = _router(x_flat, W_router, b_router)
    dest, row_token, roww, be, na = _dispatch_metadata(top_idx, top_w, nb)

    xg = _make_sc_gather(nb * BM)(x_flat, row_token)
    yw = _grouped_ffn(nb, be, na, xg, W1, b1, W2, b2, roww)
    out = _make_sc_combine(t_tokens)(yw, dest)
    return out.reshape(B, S, H)
